# baseline (speedup 1.0000x reference)
import functools

import jax
import jax.numpy as jnp
import numpy as np
from jax import lax
from jax.experimental import pallas as pl
from jax.experimental.pallas import tpu as pltpu
from jax.experimental.pallas import tpu_sc as plsc

SEQ = 8192
DIM = 512
K = 10
BLK = 512
NTILES = 16
PER_TILE = SEQ // NTILES
LANES = 16


def _pe_tables():
    dk = np.exp(
        np.arange(0, 2 * DIM, 2, dtype=np.float32) * -(np.log(10000.0) / (2 * DIM))
    )
    dk = np.repeat(dk, 2).astype(np.float64)
    u = np.arange(BLK, dtype=np.float64)[:, None]
    qb = (np.arange(SEQ // BLK, dtype=np.float64) * BLK)[:, None]
    Q = np.cos(u * dk).astype(np.float32)
    S = np.sin(u * dk).astype(np.float32)
    bsin = np.sin(qb * dk)
    bcos = np.cos(qb * dk)
    even = (np.arange(2 * DIM) % 2) == 0
    P = np.where(even, bsin, bcos).astype(np.float32)
    R = np.where(even, bcos, -bsin).astype(np.float32)
    return P, Q, R, S


XB = 4 * BLK
NPH = SEQ // XB


def _fused_body(x_ref, p_ref, q_ref, r_ref, s_ref, w1_ref,
                b1_ref, w2_ref, b2_ref, feat_ref, score_ref, uv, gacc):
    j = pl.program_id(0)
    dn = (((1,), (1,)), ((), ()))

    @pl.when(j < NPH)
    def _():
        @pl.when(j == 0)
        def _():
            gacc[...] = jnp.zeros_like(gacc)

        for h in range(XB // BLK):
            xt = x_ref[:, pl.ds(h * BLK, BLK)].T
            feat_ref[pl.ds(h * BLK, BLK), :] = xt
            pe = p_ref[h] * q_ref[...] + r_ref[h] * s_ref[...]
            u = lax.dot_general(xt + pe[:, :DIM], w1_ref[:, :DIM], dn,
                                preferred_element_type=jnp.float32)
            u = u + lax.dot_general(pe[:, DIM:], w1_ref[:, DIM:], dn,
                                    preferred_element_type=jnp.float32)
            row0 = pl.multiple_of(j * XB + h * BLK, BLK)
            uv[pl.ds(row0, BLK), :] = u + b1_ref[...]
            gacc[...] += jnp.sum(xt, axis=0, keepdims=True)

        @pl.when(j == NPH - 1)
        def _():
            gacc[...] *= 1.0 / SEQ

    @pl.when(j == NPH)
    def _():
        c = lax.dot_general(gacc[...], w1_ref[:, DIM:], dn,
                            preferred_element_type=jnp.float32)
        for jj in range(SEQ // BLK):
            row0 = pl.multiple_of(jj * BLK, BLK)
            h = jnp.maximum(uv[pl.ds(row0, BLK), :] + c, 0.0)
            s = lax.dot_general(w2_ref[...], h, dn,
                                preferred_element_type=jnp.float32)
            score_ref[0, pl.ds(row0, BLK)] = (s + b2_ref[...])[0]


def _tmerge(a, b):
    av, ai = a
    bv = lax.rev(b[0], (0,))
    bi = lax.rev(b[1], (0,))
    take = bv > av
    mv = jnp.where(take, bv, av)
    mi = jnp.where(take, bi, ai)
    return plsc.sort_key_val(mv, mi, descending=True)


def _tournament(lists):
    while len(lists) > 1:
        lists = [_tmerge(lists[2 * i], lists[2 * i + 1])
                 for i in range(len(lists) // 2)]
    return lists[0]


def _sc_body(score_hbm, feat_hbm, oh_hbm, sel_hbm,
             scv, stage_v, stage_i, idxv, candv_sh, candi_sh, fidx_sh,
             bigv, bigi, ohv, rowsv, sem):
    wid = lax.axis_index("s")
    base = pl.multiple_of(wid * PER_TILE, PER_TILE)
    pltpu.sync_copy(score_hbm.at[pl.ds(base, PER_TILE)], scv)
    lane = lax.iota(jnp.int32, LANES)

    lists = []
    for j in range(PER_TILE // LANES):
        v = scv[pl.ds(j * LANES, LANES)]
        i = base + j * LANES + lane
        lists.append(plsc.sort_key_val(v, i, descending=True))
    cand_v, cand_i = _tournament(lists)
    stage_v[...] = cand_v
    stage_i[...] = cand_i
    slot = pl.multiple_of(wid * LANES, LANES)
    pltpu.sync_copy(stage_v, candv_sh.at[pl.ds(slot, LANES)])
    pltpu.sync_copy(stage_i, candi_sh.at[pl.ds(slot, LANES)])
    plsc.subcore_barrier()

    @pl.when(wid == 0)
    def _():
        pltpu.sync_copy(candv_sh, bigv)
        pltpu.sync_copy(candi_sh, bigi)
        lists2 = [(bigv[pl.ds(j * LANES, LANES)],
                   bigi[pl.ds(j * LANES, LANES)]) for j in range(NTILES)]
        cv, ci = _tournament(lists2)
        key = jnp.where(lane < K, ci, jnp.int32(2147483647))
        si, _ = plsc.sort_key_val(key, cv)
        si = jnp.where(lane < K, si, 0)
        stage_i[...] = si
        pltpu.sync_copy(stage_i, fidx_sh)

    plsc.subcore_barrier()

    pltpu.sync_copy(fidx_sh, idxv)
    fi = idxv[...]
    col = fi - base
    valid = (col >= 0) & (col < PER_TILE) & (lane < K)
    zero16 = jnp.zeros((LANES,), jnp.float32)
    for j in range(K * PER_TILE // LANES):
        ohv[pl.ds(j * LANES, LANES)] = zero16
    off = jnp.where(valid, lane * PER_TILE + col, 0)
    plsc.store_scatter(ohv, [off], jnp.ones((LANES,), jnp.float32),
                       mask=valid)
    for r in range(K):
        pltpu.sync_copy(
            ohv.at[pl.ds(r * PER_TILE, PER_TILE)],
            oh_hbm.at[pl.ds(r * SEQ + base, PER_TILE)],
        )

    @pl.when(wid == 0)
    def _():
        pltpu.async_copy(feat_hbm.at[idxv], rowsv, sem).wait()
        pltpu.sync_copy(rowsv, sel_hbm)


def kernel(x, W1, b1, W2, b2):
    x2 = x.reshape(DIM, SEQ)
    P, Q, R, S = (jnp.asarray(t) for t in _pe_tables())
    b1r = b1.reshape(1, DIM)
    w2r = W2.reshape(1, DIM)
    b2r = b2.reshape(1, 1)

    feature, score = pl.pallas_call(
        _fused_body,
        grid=(NPH + 1,),
        in_specs=[
            pl.BlockSpec((DIM, XB), lambda j: (0, jnp.minimum(j, NPH - 1))),
            pl.BlockSpec((XB // BLK, 1, 2 * DIM),
                         lambda j: (jnp.minimum(j, NPH - 1), 0, 0)),
            pl.BlockSpec((BLK, 2 * DIM), lambda j: (0, 0)),
            pl.BlockSpec((XB // BLK, 1, 2 * DIM),
                         lambda j: (jnp.minimum(j, NPH - 1), 0, 0)),
            pl.BlockSpec((BLK, 2 * DIM), lambda j: (0, 0)),
            pl.BlockSpec((DIM, 2 * DIM), lambda j: (0, 0)),
            pl.BlockSpec((1, DIM), lambda j: (0, 0)),
            pl.BlockSpec((1, DIM), lambda j: (0, 0)),
            pl.BlockSpec((1, 1), lambda j: (0, 0)),
        ],
        out_specs=[
            pl.BlockSpec((XB, DIM), lambda j: (jnp.minimum(j, NPH - 1), 0)),
            pl.BlockSpec((1, SEQ), lambda j: (0, 0)),
        ],
        out_shape=[
            jax.ShapeDtypeStruct((SEQ, DIM), jnp.float32),
            jax.ShapeDtypeStruct((1, SEQ), jnp.float32),
        ],
        scratch_shapes=[
            pltpu.VMEM((SEQ, DIM), jnp.float32),
            pltpu.VMEM((1, DIM), jnp.float32),
        ],
    )(x2, P[:, None, :], Q, R[:, None, :], S, W1, b1r, w2r, b2r)

    score1d = score.reshape(SEQ)

    sc_fn = pl.kernel(
        _sc_body,
        out_type=[
            jax.ShapeDtypeStruct((K * SEQ,), jnp.float32),
            jax.ShapeDtypeStruct((LANES, DIM), jnp.float32),
        ],
        mesh=plsc.VectorSubcoreMesh(
            core_axis_name="c", subcore_axis_name="s", num_cores=1
        ),
        compiler_params=pltpu.CompilerParams(needs_layout_passes=False),
        scratch_types=[
            pltpu.VMEM((PER_TILE,), jnp.float32),
            pltpu.VMEM((LANES,), jnp.float32),
            pltpu.VMEM((LANES,), jnp.int32),
            pltpu.VMEM((LANES,), jnp.int32),
            pltpu.VMEM_SHARED((NTILES * LANES,), jnp.float32),
            pltpu.VMEM_SHARED((NTILES * LANES,), jnp.int32),
            pltpu.VMEM_SHARED((LANES,), jnp.int32),
            pltpu.VMEM((NTILES * LANES,), jnp.float32),
            pltpu.VMEM((NTILES * LANES,), jnp.int32),
            pltpu.VMEM((K * PER_TILE,), jnp.float32),
            pltpu.VMEM((LANES, DIM), jnp.float32),
            pltpu.SemaphoreType.DMA,
        ],
    )
    onehot, selected16 = sc_fn(score1d, feature)

    return (
        onehot.reshape(1, K, SEQ),
        selected16[:K].reshape(1, K, DIM),
        feature.reshape(1, SEQ, DIM),
    )


# speedup vs baseline: 1.0118x; 1.0118x over previous
import functools

import jax
import jax.numpy as jnp
import numpy as np
from jax import lax
from jax.experimental import pallas as pl
from jax.experimental.pallas import tpu as pltpu
from jax.experimental.pallas import tpu_sc as plsc

SEQ = 8192
DIM = 512
K = 10
BLK = 512
NTILES = 16
PER_TILE = SEQ // NTILES
LANES = 16


def _pe_tables():
    dk = np.exp(
        np.arange(0, 2 * DIM, 2, dtype=np.float32) * -(np.log(10000.0) / (2 * DIM))
    )
    dk = np.repeat(dk, 2).astype(np.float64)
    b = np.arange(32, dtype=np.float64)[:, None]
    a32 = (np.arange(BLK // 32, dtype=np.float64) * 32)[:, None]
    qb = (np.arange(SEQ // BLK, dtype=np.float64) * BLK)[:, None]
    C1 = np.cos(b * dk).astype(np.float32)
    S1 = np.sin(b * dk).astype(np.float32)
    C32 = np.cos(a32 * dk).astype(np.float32)
    S32 = np.sin(a32 * dk).astype(np.float32)
    bsin = np.sin(qb * dk)
    bcos = np.cos(qb * dk)
    even = (np.arange(2 * DIM) % 2) == 0
    P = np.where(even, bsin, bcos).astype(np.float32)
    R = np.where(even, bcos, -bsin).astype(np.float32)
    return P, R, C1, S1, C32, S32


XB = 4 * BLK
NPH = SEQ // XB


def _fused_body(x_ref, p_ref, r_ref, c1_ref, s1_ref, c32_ref, s32_ref, w1_ref,
                b1_ref, w2_ref, b2_ref, feat_ref, score_ref, uv, gacc, qv, sv):
    j = pl.program_id(0)
    dn = (((1,), (1,)), ((), ()))

    @pl.when(j < NPH)
    def _():
        @pl.when(j == 0)
        def _():
            gacc[...] = jnp.zeros_like(gacc)
            for a in range(BLK // 32):
                ca = c32_ref[pl.ds(a, 1), :]
                sa = s32_ref[pl.ds(a, 1), :]
                qv[pl.ds(a * 32, 32), :] = ca * c1_ref[...] - sa * s1_ref[...]
                sv[pl.ds(a * 32, 32), :] = sa * c1_ref[...] + ca * s1_ref[...]

        for h in range(XB // BLK):
            xt = x_ref[:, pl.ds(h * BLK, BLK)].T
            feat_ref[pl.ds(h * BLK, BLK), :] = xt
            pe = p_ref[h] * qv[...] + r_ref[h] * sv[...]
            u = lax.dot_general(xt + pe[:, :DIM], w1_ref[:, :DIM], dn,
                                preferred_element_type=jnp.float32)
            u = u + lax.dot_general(pe[:, DIM:], w1_ref[:, DIM:], dn,
                                    preferred_element_type=jnp.float32)
            row0 = pl.multiple_of(j * XB + h * BLK, BLK)
            uv[pl.ds(row0, BLK), :] = u + b1_ref[...]
            gacc[...] += jnp.sum(xt, axis=0, keepdims=True)

        @pl.when(j == NPH - 1)
        def _():
            gacc[...] *= 1.0 / SEQ

    @pl.when(j == NPH)
    def _():
        c = lax.dot_general(gacc[...], w1_ref[:, DIM:], dn,
                            preferred_element_type=jnp.float32)
        for jj in range(SEQ // BLK):
            row0 = pl.multiple_of(jj * BLK, BLK)
            h = jnp.maximum(uv[pl.ds(row0, BLK), :] + c, 0.0)
            s = lax.dot_general(w2_ref[...], h, dn,
                                preferred_element_type=jnp.float32)
            score_ref[0, pl.ds(row0, BLK)] = (s + b2_ref[...])[0]


def _tmerge(a, b):
    av, ai = a
    bv = lax.rev(b[0], (0,))
    bi = lax.rev(b[1], (0,))
    take = bv > av
    mv = jnp.where(take, bv, av)
    mi = jnp.where(take, bi, ai)
    return plsc.sort_key_val(mv, mi, descending=True)


def _tournament(lists):
    while len(lists) > 1:
        lists = [_tmerge(lists[2 * i], lists[2 * i + 1])
                 for i in range(len(lists) // 2)]
    return lists[0]


def _sc_body(score_hbm, feat_hbm, oh_hbm, sel_hbm,
             scv, stage_v, stage_i, idxv, candv_sh, candi_sh, fidx_sh,
             bigv, bigi, ohv, rowsv, sem):
    wid = lax.axis_index("s")
    base = pl.multiple_of(wid * PER_TILE, PER_TILE)
    pltpu.sync_copy(score_hbm.at[pl.ds(base, PER_TILE)], scv)
    lane = lax.iota(jnp.int32, LANES)

    lists = []
    for j in range(PER_TILE // LANES):
        v = scv[pl.ds(j * LANES, LANES)]
        i = base + j * LANES + lane
        lists.append(plsc.sort_key_val(v, i, descending=True))
    cand_v, cand_i = _tournament(lists)
    stage_v[...] = cand_v
    stage_i[...] = cand_i
    slot = pl.multiple_of(wid * LANES, LANES)
    pltpu.sync_copy(stage_v, candv_sh.at[pl.ds(slot, LANES)])
    pltpu.sync_copy(stage_i, candi_sh.at[pl.ds(slot, LANES)])
    plsc.subcore_barrier()

    @pl.when(wid == 0)
    def _():
        pltpu.sync_copy(candv_sh, bigv)
        pltpu.sync_copy(candi_sh, bigi)
        lists2 = [(bigv[pl.ds(j * LANES, LANES)],
                   bigi[pl.ds(j * LANES, LANES)]) for j in range(NTILES)]
        cv, ci = _tournament(lists2)
        key = jnp.where(lane < K, ci, jnp.int32(2147483647))
        si, _ = plsc.sort_key_val(key, cv)
        si = jnp.where(lane < K, si, 0)
        stage_i[...] = si
        pltpu.sync_copy(stage_i, fidx_sh)

    plsc.subcore_barrier()

    pltpu.sync_copy(fidx_sh, idxv)
    fi = idxv[...]
    col = fi - base
    valid = (col >= 0) & (col < PER_TILE) & (lane < K)
    zero16 = jnp.zeros((LANES,), jnp.float32)
    for j in range(K * PER_TILE // LANES):
        ohv[pl.ds(j * LANES, LANES)] = zero16
    off = jnp.where(valid, lane * PER_TILE + col, 0)
    plsc.store_scatter(ohv, [off], jnp.ones((LANES,), jnp.float32),
                       mask=valid)
    for r in range(K):
        pltpu.sync_copy(
            ohv.at[pl.ds(r * PER_TILE, PER_TILE)],
            oh_hbm.at[pl.ds(r * SEQ + base, PER_TILE)],
        )

    @pl.when(wid == 0)
    def _():
        pltpu.async_copy(feat_hbm.at[idxv], rowsv, sem).wait()
        pltpu.sync_copy(rowsv, sel_hbm)


def kernel(x, W1, b1, W2, b2):
    x2 = x.reshape(DIM, SEQ)
    P, R, C1, S1, C32, S32 = (jnp.asarray(t) for t in _pe_tables())
    b1r = b1.reshape(1, DIM)
    w2r = W2.reshape(1, DIM)
    b2r = b2.reshape(1, 1)

    feature, score = pl.pallas_call(
        _fused_body,
        grid=(NPH + 1,),
        in_specs=[
            pl.BlockSpec((DIM, XB), lambda j: (0, jnp.minimum(j, NPH - 1))),
            pl.BlockSpec((XB // BLK, 1, 2 * DIM),
                         lambda j: (jnp.minimum(j, NPH - 1), 0, 0)),
            pl.BlockSpec((XB // BLK, 1, 2 * DIM),
                         lambda j: (jnp.minimum(j, NPH - 1), 0, 0)),
            pl.BlockSpec((32, 2 * DIM), lambda j: (0, 0)),
            pl.BlockSpec((32, 2 * DIM), lambda j: (0, 0)),
            pl.BlockSpec((BLK // 32, 2 * DIM), lambda j: (0, 0)),
            pl.BlockSpec((BLK // 32, 2 * DIM), lambda j: (0, 0)),
            pl.BlockSpec((DIM, 2 * DIM), lambda j: (0, 0)),
            pl.BlockSpec((1, DIM), lambda j: (0, 0)),
            pl.BlockSpec((1, DIM), lambda j: (0, 0)),
            pl.BlockSpec((1, 1), lambda j: (0, 0)),
        ],
        out_specs=[
            pl.BlockSpec((XB, DIM), lambda j: (jnp.minimum(j, NPH - 1), 0)),
            pl.BlockSpec((1, SEQ), lambda j: (0, 0)),
        ],
        out_shape=[
            jax.ShapeDtypeStruct((SEQ, DIM), jnp.float32),
            jax.ShapeDtypeStruct((1, SEQ), jnp.float32),
        ],
        scratch_shapes=[
            pltpu.VMEM((SEQ, DIM), jnp.float32),
            pltpu.VMEM((1, DIM), jnp.float32),
            pltpu.VMEM((BLK, 2 * DIM), jnp.float32),
            pltpu.VMEM((BLK, 2 * DIM), jnp.float32),
        ],
    )(x2, P[:, None, :], R[:, None, :], C1, S1, C32, S32, W1, b1r, w2r, b2r)

    score1d = score.reshape(SEQ)

    sc_fn = pl.kernel(
        _sc_body,
        out_type=[
            jax.ShapeDtypeStruct((K * SEQ,), jnp.float32),
            jax.ShapeDtypeStruct((LANES, DIM), jnp.float32),
        ],
        mesh=plsc.VectorSubcoreMesh(
            core_axis_name="c", subcore_axis_name="s", num_cores=1
        ),
        compiler_params=pltpu.CompilerParams(needs_layout_passes=False),
        scratch_types=[
            pltpu.VMEM((PER_TILE,), jnp.float32),
            pltpu.VMEM((LANES,), jnp.float32),
            pltpu.VMEM((LANES,), jnp.int32),
            pltpu.VMEM((LANES,), jnp.int32),
            pltpu.VMEM_SHARED((NTILES * LANES,), jnp.float32),
            pltpu.VMEM_SHARED((NTILES * LANES,), jnp.int32),
            pltpu.VMEM_SHARED((LANES,), jnp.int32),
            pltpu.VMEM((NTILES * LANES,), jnp.float32),
            pltpu.VMEM((NTILES * LANES,), jnp.int32),
            pltpu.VMEM((K * PER_TILE,), jnp.float32),
            pltpu.VMEM((LANES, DIM), jnp.float32),
            pltpu.SemaphoreType.DMA,
        ],
    )
    onehot, selected16 = sc_fn(score1d, feature)

    return (
        onehot.reshape(1, K, SEQ),
        selected16[:K].reshape(1, K, DIM),
        feature.reshape(1, SEQ, DIM),
    )


# speedup vs baseline: 1.0269x; 1.0150x over previous
import functools

import jax
import jax.numpy as jnp
import numpy as np
from jax import lax
from jax.experimental import pallas as pl
from jax.experimental.pallas import tpu as pltpu
from jax.experimental.pallas import tpu_sc as plsc

SEQ = 8192
DIM = 512
K = 10
BLK = 512
NTILES = 16
PER_TILE = SEQ // NTILES
LANES = 16


def _pe_tables():
    dk = np.exp(
        np.arange(0, 2 * DIM, 2, dtype=np.float32) * -(np.log(10000.0) / (2 * DIM))
    )
    dk = np.repeat(dk, 2).astype(np.float64)
    b = np.arange(32, dtype=np.float64)[:, None]
    a32 = (np.arange(BLK // 32, dtype=np.float64) * 32)[:, None]
    qb = (np.arange(SEQ // BLK, dtype=np.float64) * BLK)[:, None]
    C1 = np.cos(b * dk).astype(np.float32)
    S1 = np.sin(b * dk).astype(np.float32)
    C32 = np.cos(a32 * dk).astype(np.float32)
    S32 = np.sin(a32 * dk).astype(np.float32)
    bsin = np.sin(qb * dk)
    bcos = np.cos(qb * dk)
    even = (np.arange(2 * DIM) % 2) == 0
    P = np.where(even, bsin, bcos).astype(np.float32)
    R = np.where(even, bcos, -bsin).astype(np.float32)
    return P, R, C1, S1, C32, S32


XB = 4 * BLK
NPH = SEQ // XB


def _fused_body(x_ref, p_ref, r_ref, c1_ref, s1_ref, c32_ref, s32_ref, w1_ref,
                b1_ref, w2_ref, b2_ref, feat_ref, score_ref, uv, gacc, qv, sv):
    j = pl.program_id(0)
    dn = (((1,), (1,)), ((), ()))

    @pl.when(j < NPH)
    def _():
        @pl.when(j == 0)
        def _():
            gacc[...] = jnp.zeros_like(gacc)
            for a in range(BLK // 32):
                ca = c32_ref[pl.ds(a, 1), :]
                sa = s32_ref[pl.ds(a, 1), :]
                qv[pl.ds(a * 32, 32), :] = ca * c1_ref[...] - sa * s1_ref[...]
                sv[pl.ds(a * 32, 32), :] = sa * c1_ref[...] + ca * s1_ref[...]

        for h in range(XB // BLK):
            xt = x_ref[:, pl.ds(h * BLK, BLK)].T
            feat_ref[pl.ds(h * BLK, BLK), :] = xt
            pe = p_ref[h] * qv[...] + r_ref[h] * sv[...]
            u = lax.dot_general(xt + pe[:, :DIM], w1_ref[:, :DIM], dn,
                                preferred_element_type=jnp.float32)
            u = u + lax.dot_general(pe[:, DIM:], w1_ref[:, DIM:], dn,
                                    preferred_element_type=jnp.float32)
            row0 = pl.multiple_of(j * XB + h * BLK, BLK)
            uv[pl.ds(row0, BLK), :] = u + b1_ref[...]
            gacc[...] += jnp.sum(xt, axis=0, keepdims=True)

        @pl.when(j == NPH - 1)
        def _():
            gacc[...] *= 1.0 / SEQ

    @pl.when(j == NPH)
    def _():
        c = lax.dot_general(gacc[...], w1_ref[:, DIM:], dn,
                            preferred_element_type=jnp.float32)
        for jj in range(SEQ // BLK):
            row0 = pl.multiple_of(jj * BLK, BLK)
            h = jnp.maximum(uv[pl.ds(row0, BLK), :] + c, 0.0)
            s = lax.dot_general(w2_ref[...], h, dn,
                                preferred_element_type=jnp.float32)
            score_ref[0, pl.ds(row0, BLK)] = (s + b2_ref[...])[0]


def _tmerge(a, b):
    av, ai = a
    bv = lax.rev(b[0], (0,))
    bi = lax.rev(b[1], (0,))
    take = bv > av
    mv = jnp.where(take, bv, av)
    mi = jnp.where(take, bi, ai)
    return plsc.sort_key_val(mv, mi, descending=True)


def _tournament(lists):
    while len(lists) > 1:
        lists = [_tmerge(lists[2 * i], lists[2 * i + 1])
                 for i in range(len(lists) // 2)]
    return lists[0]


def _sc_body(score_hbm, feat_hbm, oh_hbm, sel_hbm,
             scv, stage_v, stage_i, idxv, candv_sh, candi_sh, fidx_sh,
             bigv, bigi, obuf, rowsv, sem):
    wid = lax.axis_index("s")
    base = pl.multiple_of(wid * PER_TILE, PER_TILE)
    desc = pltpu.async_copy(score_hbm.at[pl.ds(base, PER_TILE)], scv, sem)
    lane = lax.iota(jnp.int32, LANES)
    zero16 = jnp.zeros((LANES,), jnp.float32)

    @pl.when(wid < K)
    def _():
        for jz in range(SEQ // LANES):
            obuf[pl.ds(jz * LANES, LANES)] = zero16

    desc.wait()

    lists = []
    for j in range(PER_TILE // LANES):
        v = scv[pl.ds(j * LANES, LANES)]
        i = base + j * LANES + lane
        lists.append(plsc.sort_key_val(v, i, descending=True))
    cand_v, cand_i = _tournament(lists)
    stage_v[...] = cand_v
    stage_i[...] = cand_i
    slot = pl.multiple_of(wid * LANES, LANES)
    pltpu.sync_copy(stage_v, candv_sh.at[pl.ds(slot, LANES)])
    pltpu.sync_copy(stage_i, candi_sh.at[pl.ds(slot, LANES)])
    plsc.subcore_barrier()

    @pl.when(wid == 0)
    def _():
        pltpu.sync_copy(candv_sh, bigv)
        pltpu.sync_copy(candi_sh, bigi)
        lists2 = [(bigv[pl.ds(j * LANES, LANES)],
                   bigi[pl.ds(j * LANES, LANES)]) for j in range(NTILES)]
        cv, ci = _tournament(lists2)
        key = jnp.where(lane < K, ci, jnp.int32(2147483647))
        si, _ = plsc.sort_key_val(key, cv)
        si = jnp.where(lane < K, si, 0)
        stage_i[...] = si
        pltpu.sync_copy(stage_i, fidx_sh)

    plsc.subcore_barrier()

    pltpu.sync_copy(fidx_sh, idxv)
    fi = idxv[...]

    @pl.when(wid < K)
    def _():
        plsc.store_scatter(obuf, [fi], jnp.ones((LANES,), jnp.float32),
                           mask=lane == wid)
        rowoff = pl.multiple_of(wid * SEQ, SEQ)
        pltpu.sync_copy(obuf, oh_hbm.at[pl.ds(rowoff, SEQ)])

    @pl.when(wid == NTILES - 1)
    def _():
        pltpu.async_copy(feat_hbm.at[idxv], rowsv, sem).wait()
        pltpu.sync_copy(rowsv, sel_hbm)


def kernel(x, W1, b1, W2, b2):
    x2 = x.reshape(DIM, SEQ)
    P, R, C1, S1, C32, S32 = (jnp.asarray(t) for t in _pe_tables())
    b1r = b1.reshape(1, DIM)
    w2r = W2.reshape(1, DIM)
    b2r = b2.reshape(1, 1)

    feature, score = pl.pallas_call(
        _fused_body,
        grid=(NPH + 1,),
        in_specs=[
            pl.BlockSpec((DIM, XB), lambda j: (0, jnp.minimum(j, NPH - 1))),
            pl.BlockSpec((XB // BLK, 1, 2 * DIM),
                         lambda j: (jnp.minimum(j, NPH - 1), 0, 0)),
            pl.BlockSpec((XB // BLK, 1, 2 * DIM),
                         lambda j: (jnp.minimum(j, NPH - 1), 0, 0)),
            pl.BlockSpec((32, 2 * DIM), lambda j: (0, 0)),
            pl.BlockSpec((32, 2 * DIM), lambda j: (0, 0)),
            pl.BlockSpec((BLK // 32, 2 * DIM), lambda j: (0, 0)),
            pl.BlockSpec((BLK // 32, 2 * DIM), lambda j: (0, 0)),
            pl.BlockSpec((DIM, 2 * DIM), lambda j: (0, 0)),
            pl.BlockSpec((1, DIM), lambda j: (0, 0)),
            pl.BlockSpec((1, DIM), lambda j: (0, 0)),
            pl.BlockSpec((1, 1), lambda j: (0, 0)),
        ],
        out_specs=[
            pl.BlockSpec((XB, DIM), lambda j: (jnp.minimum(j, NPH - 1), 0)),
            pl.BlockSpec((1, SEQ), lambda j: (0, 0)),
        ],
        out_shape=[
            jax.ShapeDtypeStruct((SEQ, DIM), jnp.float32),
            jax.ShapeDtypeStruct((1, SEQ), jnp.float32),
        ],
        scratch_shapes=[
            pltpu.VMEM((SEQ, DIM), jnp.float32),
            pltpu.VMEM((1, DIM), jnp.float32),
            pltpu.VMEM((BLK, 2 * DIM), jnp.float32),
            pltpu.VMEM((BLK, 2 * DIM), jnp.float32),
        ],
    )(x2, P[:, None, :], R[:, None, :], C1, S1, C32, S32, W1, b1r, w2r, b2r)

    score1d = score.reshape(SEQ)

    sc_fn = pl.kernel(
        _sc_body,
        out_type=[
            jax.ShapeDtypeStruct((K * SEQ,), jnp.float32),
            jax.ShapeDtypeStruct((LANES, DIM), jnp.float32),
        ],
        mesh=plsc.VectorSubcoreMesh(
            core_axis_name="c", subcore_axis_name="s", num_cores=1
        ),
        compiler_params=pltpu.CompilerParams(needs_layout_passes=False),
        scratch_types=[
            pltpu.VMEM((PER_TILE,), jnp.float32),
            pltpu.VMEM((LANES,), jnp.float32),
            pltpu.VMEM((LANES,), jnp.int32),
            pltpu.VMEM((LANES,), jnp.int32),
            pltpu.VMEM_SHARED((NTILES * LANES,), jnp.float32),
            pltpu.VMEM_SHARED((NTILES * LANES,), jnp.int32),
            pltpu.VMEM_SHARED((LANES,), jnp.int32),
            pltpu.VMEM((NTILES * LANES,), jnp.float32),
            pltpu.VMEM((NTILES * LANES,), jnp.int32),
            pltpu.VMEM((SEQ,), jnp.float32),
            pltpu.VMEM((LANES, DIM), jnp.float32),
            pltpu.SemaphoreType.DMA,
        ],
    )
    onehot, selected16 = sc_fn(score1d, feature)

    return (
        onehot.reshape(1, K, SEQ),
        selected16[:K].reshape(1, K, DIM),
        feature.reshape(1, SEQ, DIM),
    )
